# baseline (speedup 1.0000x reference)
import functools

import jax
import jax.numpy as jnp
from jax.experimental import pallas as pl
from jax.experimental.pallas import tpu as pltpu

B = 128
H = 64


def _body(users_ref, items_ref, um_hbm, im_hbm,
          wih_u, whh_u, bias_u, wih_i, whh_i, bias_i,
          ue_out, ie_out, um_out, im_out,
          ue_v, ie_v, sem_g, sem_s):
    for b in range(B):
        pltpu.make_async_copy(um_hbm.at[b, users_ref[b]], ue_v.at[b], sem_g).start()
        pltpu.make_async_copy(im_hbm.at[b, items_ref[b]], ie_v.at[b], sem_g).start()
    for b in range(B):
        pltpu.make_async_copy(um_hbm.at[b, users_ref[b]], ue_v.at[b], sem_g).wait()
        pltpu.make_async_copy(im_hbm.at[b, items_ref[b]], ie_v.at[b], sem_g).wait()

    ue = ue_v[...]
    ie = ie_v[...]

    def gru(x, h, wih, whh, bias):
        g = (jnp.dot(x, wih, preferred_element_type=jnp.float32)
             + jnp.dot(h, whh, preferred_element_type=jnp.float32)
             + bias)
        gi = g[:, :3 * H]
        gh = g[:, 3 * H:]
        r = jax.nn.sigmoid(gi[:, :H] + gh[:, :H])
        z = jax.nn.sigmoid(gi[:, H:2 * H] + gh[:, H:2 * H])
        n = jnp.tanh(gi[:, 2 * H:] + r * gh[:, 2 * H:])
        return (1.0 - z) * n + z * h

    x_u = jnp.concatenate([ue, ie], axis=1)
    x_i = jnp.concatenate([ie, ue], axis=1)
    ue_out[...] = gru(x_u, ue, wih_u[...], whh_u[...], bias_u[...])
    ie_out[...] = gru(x_i, ie, wih_i[...], whh_i[...], bias_i[...])

    for b in range(B):
        pltpu.make_async_copy(ue_out.at[b], um_out.at[b, users_ref[b]], sem_s).start()
        pltpu.make_async_copy(ie_out.at[b], im_out.at[b, items_ref[b]], sem_s).start()
    for b in range(B):
        pltpu.make_async_copy(ue_out.at[b], um_out.at[b, users_ref[b]], sem_s).wait()
        pltpu.make_async_copy(ie_out.at[b], im_out.at[b, items_ref[b]], sem_s).wait()


def kernel(user_memory, item_memory, users, items,
           W_ih_u, W_hh_u, b_ih_u, b_hh_u,
           W_ih_i, W_hh_i, b_ih_i, b_hh_i):
    users = users.astype(jnp.int32)
    items = items.astype(jnp.int32)
    wih_u2 = jnp.concatenate([W_ih_u.T, jnp.zeros((2 * H, 3 * H), jnp.float32)], axis=1)
    whh_u2 = jnp.concatenate([jnp.zeros((H, 3 * H), jnp.float32), W_hh_u.T], axis=1)
    wih_i2 = jnp.concatenate([W_ih_i.T, jnp.zeros((2 * H, 3 * H), jnp.float32)], axis=1)
    whh_i2 = jnp.concatenate([jnp.zeros((H, 3 * H), jnp.float32), W_hh_i.T], axis=1)
    bias_u2 = jnp.concatenate([b_ih_u, b_hh_u]).reshape(1, 6 * H)
    bias_i2 = jnp.concatenate([b_ih_i, b_hh_i]).reshape(1, 6 * H)

    out_shape = (
        jax.ShapeDtypeStruct((B, H), jnp.float32),
        jax.ShapeDtypeStruct((B, H), jnp.float32),
        jax.ShapeDtypeStruct(user_memory.shape, jnp.float32),
        jax.ShapeDtypeStruct(item_memory.shape, jnp.float32),
    )
    grid_spec = pltpu.PrefetchScalarGridSpec(
        num_scalar_prefetch=2,
        grid=(),
        in_specs=[
            pl.BlockSpec(memory_space=pl.ANY),
            pl.BlockSpec(memory_space=pl.ANY),
            pl.BlockSpec(memory_space=pltpu.VMEM),
            pl.BlockSpec(memory_space=pltpu.VMEM),
            pl.BlockSpec(memory_space=pltpu.VMEM),
            pl.BlockSpec(memory_space=pltpu.VMEM),
            pl.BlockSpec(memory_space=pltpu.VMEM),
            pl.BlockSpec(memory_space=pltpu.VMEM),
        ],
        out_specs=[
            pl.BlockSpec(memory_space=pltpu.VMEM),
            pl.BlockSpec(memory_space=pltpu.VMEM),
            pl.BlockSpec(memory_space=pl.ANY),
            pl.BlockSpec(memory_space=pl.ANY),
        ],
        scratch_shapes=[
            pltpu.VMEM((B, H), jnp.float32),
            pltpu.VMEM((B, H), jnp.float32),
            pltpu.SemaphoreType.DMA,
            pltpu.SemaphoreType.DMA,
        ],
    )
    ue, ie, new_um, new_im = pl.pallas_call(
        _body,
        grid_spec=grid_spec,
        out_shape=out_shape,
        input_output_aliases={2: 2, 3: 3},
        name="limnet_step",
    )(users, items, user_memory, item_memory,
      wih_u2, whh_u2, bias_u2, wih_i2, whh_i2, bias_i2)
    return (ue, ie, new_um, new_im)
